# Initial kernel scaffold; baseline (speedup 1.0000x reference)
import functools

import jax
import jax.numpy as jnp
from jax.experimental import pallas as pl
from jax.experimental.pallas import tpu as pltpu


def _crop_mul_body(pos_ref, obj_ref, waves_ref, out_ref, pn_ref):
    b = pl.program_id(0)
    r = pos_ref[b, 0]
    c = pos_ref[b, 1]
    patch = obj_ref[0, pl.ds(r, 128), pl.ds(c, 128)]
    w = waves_ref[0, 0]
    out_ref[0, 0] = w * patch

    @pl.when(b == 0)
    def _init():
        pn_ref[...] = jnp.zeros_like(pn_ref)

    pn_ref[...] += w * w


def _scatter_body(pos_ref, pn_ref, on_ref):
    b = pl.program_id(0)

    @pl.when(b == 0)
    def _init():
        on_ref[...] = jnp.zeros_like(on_ref)

    r = pos_ref[b, 0]
    c = pos_ref[b, 1]
    on_ref[pl.ds(r, 128), pl.ds(c, 128)] += pn_ref[...]


@jax.jit
def kernel(obj, waves, pos):
    B = waves.shape[0]
    h, w = waves.shape[-2], waves.shape[-1]
    H, W = obj.shape[-2], obj.shape[-1]
    pos32 = pos.astype(jnp.int32)

    grid_spec = pltpu.PrefetchScalarGridSpec(
        num_scalar_prefetch=1,
        grid=(B,),
        in_specs=[
            pl.BlockSpec((1, H, W), lambda b, p: (0, 0, 0)),
            pl.BlockSpec((1, 1, h, w), lambda b, p: (b, 0, 0, 0)),
        ],
        out_specs=[
            pl.BlockSpec((1, 1, h, w), lambda b, p: (b, 0, 0, 0)),
            pl.BlockSpec((h, w), lambda b, p: (0, 0)),
        ],
    )
    out, patch_norm = pl.pallas_call(
        _crop_mul_body,
        grid_spec=grid_spec,
        out_shape=[
            jax.ShapeDtypeStruct((B, 1, h, w), jnp.float32),
            jax.ShapeDtypeStruct((h, w), jnp.float32),
        ],
    )(pos32, obj, waves)

    scatter_spec = pltpu.PrefetchScalarGridSpec(
        num_scalar_prefetch=1,
        grid=(B,),
        in_specs=[pl.BlockSpec((h, w), lambda b, p: (0, 0))],
        out_specs=pl.BlockSpec((H, W), lambda b, p: (0, 0)),
    )
    object_norm = pl.pallas_call(
        _scatter_body,
        grid_spec=scatter_spec,
        out_shape=jax.ShapeDtypeStruct((H, W), jnp.float32),
    )(pos32, patch_norm)

    return (out, object_norm)


# speedup vs baseline: 37.6490x; 37.6490x over previous
import functools

import jax
import jax.numpy as jnp
from jax.experimental import pallas as pl
from jax.experimental.pallas import tpu as pltpu

_H = 128
_BR = _H + 8
_BC = 2 * _H


def _crop_mul_body(pos_ref, obj_ref, waves_ref, out_ref, pn_ref):
    b = pl.program_id(0)
    r = pos_ref[b, 0]
    c = pos_ref[b, 1]
    r8 = (r // 8) * 8
    t = r - r8
    c128 = (c // _H) * _H
    s = c - c128
    big = obj_ref[0, pl.ds(r8, _BR), pl.ds(c128, _BC)]
    rolled = pltpu.roll(pltpu.roll(big, _BR - t, axis=0), _BC - s, axis=1)
    patch = rolled[: _H, : _H]
    w = waves_ref[0, 0]
    out_ref[0, 0] = w * patch

    @pl.when(b == 0)
    def _init():
        pn_ref[...] = jnp.zeros_like(pn_ref)

    pn_ref[...] += w * w


def _scatter_body(pos_ref, pn_ref, on_ref):
    b = pl.program_id(0)

    @pl.when(b == 0)
    def _init():
        on_ref[...] = jnp.zeros_like(on_ref)

    r = pos_ref[b, 0]
    c = pos_ref[b, 1]
    r8 = (r // 8) * 8
    t = r - r8
    c128 = (c // _H) * _H
    s = c - c128
    pn = pn_ref[...]
    padded = jnp.pad(pn, ((0, _BR - _H), (0, _BC - _H)))
    placed = pltpu.roll(pltpu.roll(padded, t, axis=0), s, axis=1)
    on_ref[pl.ds(r8, _BR), pl.ds(c128, _BC)] += placed


@jax.jit
def kernel(obj, waves, pos):
    B = waves.shape[0]
    h, w = waves.shape[-2], waves.shape[-1]
    H, W = obj.shape[-2], obj.shape[-1]
    pos32 = pos.astype(jnp.int32)

    grid_spec = pltpu.PrefetchScalarGridSpec(
        num_scalar_prefetch=1,
        grid=(B,),
        in_specs=[
            pl.BlockSpec((1, H, W), lambda b, p: (0, 0, 0)),
            pl.BlockSpec((1, 1, h, w), lambda b, p: (b, 0, 0, 0)),
        ],
        out_specs=[
            pl.BlockSpec((1, 1, h, w), lambda b, p: (b, 0, 0, 0)),
            pl.BlockSpec((h, w), lambda b, p: (0, 0)),
        ],
    )
    out, patch_norm = pl.pallas_call(
        _crop_mul_body,
        grid_spec=grid_spec,
        out_shape=[
            jax.ShapeDtypeStruct((B, 1, h, w), jnp.float32),
            jax.ShapeDtypeStruct((h, w), jnp.float32),
        ],
    )(pos32, obj, waves)

    scatter_spec = pltpu.PrefetchScalarGridSpec(
        num_scalar_prefetch=1,
        grid=(B,),
        in_specs=[pl.BlockSpec((h, w), lambda b, p: (0, 0))],
        out_specs=pl.BlockSpec((H, W), lambda b, p: (0, 0)),
    )
    object_norm = pl.pallas_call(
        _scatter_body,
        grid_spec=scatter_spec,
        out_shape=jax.ShapeDtypeStruct((H, W), jnp.float32),
    )(pos32, patch_norm)

    return (out, object_norm)


# speedup vs baseline: 39.1877x; 1.0409x over previous
import functools

import jax
import jax.numpy as jnp
from jax import lax
from jax.experimental import pallas as pl
from jax.experimental.pallas import tpu as pltpu
from jax.experimental.pallas import tpu_sc as plsc

_P = 128
_BR = _P + 8
_BC = 2 * _P
_H = 1024
_NB = 8
_NTILES = 32
_BAND = _H // _NTILES
_L = 16


def _pn_body(waves_ref, pn_ref):
    i = pl.program_id(0)

    @pl.when(i == 0)
    def _init():
        pn_ref[...] = jnp.zeros_like(pn_ref)

    w = waves_ref[:, 0]
    pn_ref[...] += jnp.sum(w * w, axis=0)


def _crop_mul_body(pos_ref, obj_ref, waves_ref, out_ref):
    b = pl.program_id(0)
    r = pos_ref[b, 0]
    c = pos_ref[b, 1]
    r8 = (r // 8) * 8
    t = r - r8
    c128 = (c // _P) * _P
    s = c - c128
    big = obj_ref[0, pl.ds(r8, _BR), pl.ds(c128, _BC)]
    rolled = pltpu.roll(pltpu.roll(big, _BR - t, axis=0), _BC - s, axis=1)
    out_ref[0, 0] = waves_ref[0, 0] * rolled[: _P, : _P]


def _sc_scatter_body(pn_hbm, posr_hbm, posc_hbm, on_hbm,
                     pn_v, posr_v, posc_v, band_v):
    cid = lax.axis_index("c")
    sid = lax.axis_index("s")
    wid = sid * 2 + cid
    y0 = wid * _BAND

    pltpu.sync_copy(pn_hbm, pn_v)
    pltpu.sync_copy(posr_hbm, posr_v)
    pltpu.sync_copy(posc_hbm, posc_v)

    zeros = jnp.zeros((_L,), jnp.float32)

    def zrow(i, _):
        def zcol(j, _):
            band_v[i, pl.ds(pl.multiple_of(j * _L, _L), _L)] = zeros
            return 0
        lax.fori_loop(0, _H // _L, zcol, 0)
        return 0
    lax.fori_loop(0, _BAND, zrow, 0)

    lane = lax.iota(jnp.int32, _L)

    def chunk_body(cb, _):
        p0v = posr_v[pl.ds(pl.multiple_of(cb * _L, _L), _L)]
        p1v = posc_v[pl.ds(pl.multiple_of(cb * _L, _L), _L)]
        for k in range(_L):
            p0 = p0v[k]
            p1 = p1v[k]
            lo = jnp.maximum(p0, y0)
            hi = jnp.minimum(p0 + _P, y0 + _BAND)
            jlo = p1 // _L
            base_rel = jlo * _L - p1

            def row_body(y, _, p0=p0, jlo=jlo, base_rel=base_rel):
                rb = (y - p0) * _P
                yloc = y - y0
                for jj in range(9):
                    rel = base_rel + jj * _L + lane
                    valid = (rel >= 0) & (rel < _P)
                    idx = rb + jnp.clip(rel, 0, _P - 1)
                    val = plsc.load_gather(pn_v, [idx])
                    val = jnp.where(valid, val, 0.0)
                    cs = pl.multiple_of((jlo + jj) * _L, _L)
                    band_v[yloc, pl.ds(cs, _L)] += val
                return 0

            lax.fori_loop(lo, hi, row_body, 0)
        return 0

    lax.fori_loop(0, 256 // _L, chunk_body, 0)
    pltpu.sync_copy(band_v, on_hbm.at[pl.ds(y0, _BAND)])


def _sc_scatter(pn_flat, pos_r, pos_c):
    mesh = plsc.VectorSubcoreMesh(
        core_axis_name="c", subcore_axis_name="s", num_cores=2, num_subcores=16
    )
    return pl.kernel(
        _sc_scatter_body,
        out_type=jax.ShapeDtypeStruct((_H, _H), jnp.float32),
        mesh=mesh,
        compiler_params=pltpu.CompilerParams(needs_layout_passes=False),
        scratch_types=[
            pltpu.VMEM((_P * _P,), jnp.float32),
            pltpu.VMEM((256,), jnp.int32),
            pltpu.VMEM((256,), jnp.int32),
            pltpu.VMEM((_BAND, _H), jnp.float32),
        ],
    )(pn_flat, pos_r, pos_c)


@jax.jit
def kernel(obj, waves, pos):
    B = waves.shape[0]
    h, w = waves.shape[-2], waves.shape[-1]
    H, W = obj.shape[-2], obj.shape[-1]
    pos32 = pos.astype(jnp.int32)

    patch_norm = pl.pallas_call(
        _pn_body,
        grid=(B // _NB,),
        in_specs=[pl.BlockSpec((_NB, 1, h, w), lambda i: (i, 0, 0, 0))],
        out_specs=pl.BlockSpec((h, w), lambda i: (0, 0)),
        out_shape=jax.ShapeDtypeStruct((h, w), jnp.float32),
    )(waves)

    object_norm = _sc_scatter(
        patch_norm.reshape(h * w),
        pos32[:, 0],
        pos32[:, 1],
    )

    grid_spec = pltpu.PrefetchScalarGridSpec(
        num_scalar_prefetch=1,
        grid=(B,),
        in_specs=[
            pl.BlockSpec((1, H, W), lambda b, p: (0, 0, 0)),
            pl.BlockSpec((1, 1, h, w), lambda b, p: (b, 0, 0, 0)),
        ],
        out_specs=pl.BlockSpec((1, 1, h, w), lambda b, p: (b, 0, 0, 0)),
    )
    out = pl.pallas_call(
        _crop_mul_body,
        grid_spec=grid_spec,
        out_shape=jax.ShapeDtypeStruct((B, 1, h, w), jnp.float32),
    )(pos32, obj, waves)

    return (out, object_norm)
